# Initial kernel scaffold; baseline (speedup 1.0000x reference)
import jax
import jax.numpy as jnp
from jax.experimental import pallas as pl


def kernel(x, embed, W_e1, b_e1, W_e2, b_e2, W_e3, b_e3, W_d1, b_d1, W_d2, b_d2, W_d3, b_d3):
    raise NotImplementedError("write your pallas kernel here")


# speedup vs baseline: 1.1544x; 1.1544x over previous
import jax
import jax.numpy as jnp
from jax.experimental import pallas as pl
from jax.experimental.pallas import tpu as pltpu
from jax.experimental.pallas import tpu_sc as plsc

B, D_IN, H, L, K = 16384, 128, 256, 32, 8192
BM = 1024
NB = B // BM
KC = 2048
NC = K // KC
GW = 256


def _f32(c):
    return jnp.float32(c)


def _gelu(x):
    y = (-x) * _f32(0.707106769)
    ax = jnp.abs(y)
    z = y * y
    pa = z * _f32(7.85386146e-05) + _f32(-0.000801019371)
    pa = pa * z + _f32(0.00518832775)
    pa = pa * z + _f32(-0.0268538129)
    pa = pa * z + _f32(0.112835854)
    pa = pa * z + _f32(-0.37612626)
    pa = pa * z + _f32(1.12837911)
    res_a = _f32(1.0) - y * pa
    nz = -z
    e_exp = jnp.exp(nz)
    em = e_exp * (_f32(1.0) / ax)
    q = _f32(1.0) / z
    p1 = q * _f32(0.0232682) + _f32(-0.138703942)
    p1 = p1 * q + _f32(0.368742466)
    p1 = p1 * q + _f32(-0.582473278)
    p1 = p1 * q + _f32(0.621000469)
    p1 = p1 * q + _f32(-0.494451523)
    p1 = p1 * q + _f32(0.340488)
    p1 = p1 * q + _f32(-0.274112701)
    p1 = p1 * q + _f32(0.563825965)
    p2 = q * _f32(-10.477664) + _f32(12.9772)
    p2 = p2 * q + _f32(-7.49551868)
    p2 = p2 * q + _f32(2.92101908)
    p2 = p2 * q + _f32(-1.01526523)
    p2 = p2 * q + _f32(0.42184633)
    p2 = p2 * q + _f32(-0.282076746)
    p2 = p2 * q + _f32(0.564189494)
    t = em * jnp.where(ax < _f32(2.0), p1, p2)
    t = jnp.where(nz < _f32(-88.7228394), _f32(0.0), t)
    res_b = jnp.where(y < _f32(0.0), _f32(2.0) - t, t)
    erfc_y = jnp.where(ax < _f32(1.0), res_a, res_b)
    return (x * _f32(0.5)) * erfc_y


def _dot_bf(a, b):
    return jnp.dot(a.astype(jnp.bfloat16), b.astype(jnp.bfloat16),
                   preferred_element_type=jnp.float32)


def _rowsq_sum(v, axis):
    sq = v * v
    if axis == 1:
        a = ((sq[:, 0:8] + sq[:, 8:16]) + sq[:, 16:24]) + sq[:, 24:32]
        c = lambda i: a[:, i:i + 1]
    else:
        a = ((sq[0:8, :] + sq[8:16, :]) + sq[16:24, :]) + sq[24:32, :]
        c = lambda i: a[i:i + 1, :]
    return (((c(0) + c(4)) + (c(2) + c(6)))
            + ((c(1) + c(5)) + (c(3) + c(7))))


def _enc_body(x_ref, we1_ref, be1_ref, we2_ref, be2_ref, we3_ref, be3_ref,
              embt_ref, ze_ref, idx_ref):
    g1 = _gelu(_dot_bf(x_ref[...], we1_ref[...]) + be1_ref[...])
    g2 = _gelu(_dot_bf(g1, we2_ref[...]) + be2_ref[...])
    ze = _dot_bf(g2, we3_ref[...]) + be3_ref[...]
    ze_ref[...] = ze
    zsq = _rowsq_sum(ze, axis=1)
    zeb = ze.astype(jnp.bfloat16)

    tms, tis = [], []
    for t in range(NC):
        w = embt_ref[:, t * KC:(t + 1) * KC]
        esq = _rowsq_sum(w, axis=0)
        dot = jnp.dot(zeb, w.astype(jnp.bfloat16),
                      preferred_element_type=jnp.float32)
        dist = (zsq + esq) - _f32(2.0) * dot
        tmin = jnp.min(dist, axis=1, keepdims=True)
        iota = jax.lax.broadcasted_iota(jnp.int32, (BM, KC), 1) + (t * KC)
        tidx = jnp.min(jnp.where(dist == tmin, iota, jnp.int32(K)),
                       axis=1, keepdims=True)
        tms.append(tmin)
        tis.append(tidx)
    a_take1 = tms[1] < tms[0]
    a_v = jnp.where(a_take1, tms[1], tms[0])
    a_i = jnp.where(a_take1, tis[1], tis[0])
    b_take1 = tms[3] < tms[2]
    b_v = jnp.where(b_take1, tms[3], tms[2])
    b_i = jnp.where(b_take1, tis[3], tis[2])
    take_b = b_v < a_v.astype(jnp.bfloat16).astype(jnp.float32)
    idx_ref[...] = jnp.where(take_b, b_i, a_i)


def _encode_argmin(x, we1, be1, we2, be2, we3, be3, embt):
    return pl.pallas_call(
        _enc_body,
        grid=(NB,),
        in_specs=[
            pl.BlockSpec((BM, D_IN), lambda i: (i, 0)),
            pl.BlockSpec((D_IN, H), lambda i: (0, 0)),
            pl.BlockSpec((1, H), lambda i: (0, 0)),
            pl.BlockSpec((H, H), lambda i: (0, 0)),
            pl.BlockSpec((1, H), lambda i: (0, 0)),
            pl.BlockSpec((H, L), lambda i: (0, 0)),
            pl.BlockSpec((1, L), lambda i: (0, 0)),
            pl.BlockSpec((L, K), lambda i: (0, 0)),
        ],
        out_specs=[
            pl.BlockSpec((BM, L), lambda i: (i, 0)),
            pl.BlockSpec((BM, 1), lambda i: (i, 0)),
        ],
        out_shape=[
            jax.ShapeDtypeStruct((B, L), jnp.float32),
            jax.ShapeDtypeStruct((B, 1), jnp.int32),
        ],
    )(x, we1, be1, we2, be2, we3, be3, embt)


def _sc_gather(embed_pad, idx2):
    mesh = plsc.VectorSubcoreMesh(core_axis_name="core",
                                  subcore_axis_name="subcore")

    @pl.kernel(out_type=jax.ShapeDtypeStruct((B, 128), jnp.float32), mesh=mesh)
    def gather_kernel(emb_hbm, i_hbm, o_hbm):
        def body(i_vmem, o_vmem):
            pltpu.sync_copy(emb_hbm.at[i_vmem.at[0]], o_vmem)

        pltpu.emit_pipeline(
            body,
            grid=(B // GW,),
            in_specs=[pl.BlockSpec((1, GW), index_map=lambda i: (0, i))],
            out_specs=[pl.BlockSpec((GW, 128), index_map=lambda i: (i, 0))],
            core_axis_name=("core", "subcore"),
            dimension_semantics=(pltpu.PARALLEL,),
        )(i_hbm, o_hbm)

    return gather_kernel(embed_pad, idx2)


def _dec_body(zq_ref, ze_ref, idx_in_ref, wd1_ref, bd1_ref, wd2_ref, bd2_ref,
              wd3_ref, bd3_ref, xr_ref, part_ref, idx_out_ref):
    zq = zq_ref[:, :L]
    d = _gelu(_dot_bf(zq, wd1_ref[...]) + bd1_ref[...])
    d = _gelu(_dot_bf(d, wd2_ref[...]) + bd2_ref[...])
    xr_ref[...] = _dot_bf(d, wd3_ref[...]) + bd3_ref[...]
    diff = ze_ref[...] - zq
    part_ref[...] = jnp.sum(diff * diff, axis=0, keepdims=True)[None]
    idx_out_ref[...] = idx_in_ref[...]


def _decode(zq, ze, idx2, wd1, bd1, wd2, bd2, wd3, bd3):
    return pl.pallas_call(
        _dec_body,
        grid=(NB,),
        in_specs=[
            pl.BlockSpec((BM, 128), lambda i: (i, 0)),
            pl.BlockSpec((BM, L), lambda i: (i, 0)),
            pl.BlockSpec((BM, 1), lambda i: (i, 0)),
            pl.BlockSpec((L, H), lambda i: (0, 0)),
            pl.BlockSpec((1, H), lambda i: (0, 0)),
            pl.BlockSpec((H, H), lambda i: (0, 0)),
            pl.BlockSpec((1, H), lambda i: (0, 0)),
            pl.BlockSpec((H, D_IN), lambda i: (0, 0)),
            pl.BlockSpec((1, D_IN), lambda i: (0, 0)),
        ],
        out_specs=[
            pl.BlockSpec((BM, D_IN), lambda i: (i, 0)),
            pl.BlockSpec((1, 1, L), lambda i: (i, 0, 0)),
            pl.BlockSpec((BM, 1), lambda i: (i, 0)),
        ],
        out_shape=[
            jax.ShapeDtypeStruct((B, D_IN), jnp.float32),
            jax.ShapeDtypeStruct((NB, 1, L), jnp.float32),
            jax.ShapeDtypeStruct((B, 1), jnp.int32),
        ],
    )(zq, ze, idx2, wd1, bd1, wd2, bd2, wd3, bd3)


def kernel(x, embed, W_e1, b_e1, W_e2, b_e2, W_e3, b_e3,
           W_d1, b_d1, W_d2, b_d2, W_d3, b_d3):
    embt = jnp.transpose(embed)
    ze, idx2 = _encode_argmin(
        x, W_e1, b_e1.reshape(1, H), W_e2, b_e2.reshape(1, H),
        W_e3, b_e3.reshape(1, L), embt)
    embed_pad = jnp.pad(embed, ((0, 0), (0, 128 - L)))
    zq = _sc_gather(embed_pad, idx2.reshape(1, B))
    x_recon, parts, idx_out = _decode(
        zq, ze, idx2, W_d1, b_d1.reshape(1, H), W_d2, b_d2.reshape(1, H),
        W_d3, b_d3.reshape(1, D_IN))
    vq_loss = jnp.sum(parts) * (1.25 / (B * L))
    return (x_recon, vq_loss, idx_out.reshape(B))


# speedup vs baseline: 1.1613x; 1.0060x over previous
import jax
import jax.numpy as jnp
from jax.experimental import pallas as pl
from jax.experimental.pallas import tpu as pltpu
from jax.experimental.pallas import tpu_sc as plsc

B, D_IN, H, L, K = 16384, 128, 256, 32, 8192
BM = 1024
NB = B // BM
KC = 2048
NC = K // KC
GW = 256


def _f32(c):
    return jnp.float32(c)


def _gelu(x):
    y = (-x) * _f32(0.707106769)
    ax = jnp.abs(y)
    z = y * y
    pa = z * _f32(7.85386146e-05) + _f32(-0.000801019371)
    pa = pa * z + _f32(0.00518832775)
    pa = pa * z + _f32(-0.0268538129)
    pa = pa * z + _f32(0.112835854)
    pa = pa * z + _f32(-0.37612626)
    pa = pa * z + _f32(1.12837911)
    res_a = _f32(1.0) - y * pa
    nz = -z
    e_exp = jnp.exp(nz)
    em = e_exp * (_f32(1.0) / ax)
    q = _f32(1.0) / z
    p1 = q * _f32(0.0232682) + _f32(-0.138703942)
    p1 = p1 * q + _f32(0.368742466)
    p1 = p1 * q + _f32(-0.582473278)
    p1 = p1 * q + _f32(0.621000469)
    p1 = p1 * q + _f32(-0.494451523)
    p1 = p1 * q + _f32(0.340488)
    p1 = p1 * q + _f32(-0.274112701)
    p1 = p1 * q + _f32(0.563825965)
    p2 = q * _f32(-10.477664) + _f32(12.9772)
    p2 = p2 * q + _f32(-7.49551868)
    p2 = p2 * q + _f32(2.92101908)
    p2 = p2 * q + _f32(-1.01526523)
    p2 = p2 * q + _f32(0.42184633)
    p2 = p2 * q + _f32(-0.282076746)
    p2 = p2 * q + _f32(0.564189494)
    t = em * jnp.where(ax < _f32(2.0), p1, p2)
    t = jnp.where(nz < _f32(-88.7228394), _f32(0.0), t)
    res_b = jnp.where(y < _f32(0.0), _f32(2.0) - t, t)
    erfc_y = jnp.where(ax < _f32(1.0), res_a, res_b)
    return (x * _f32(0.5)) * erfc_y


def _dot_bf(a, b):
    return jnp.dot(a.astype(jnp.bfloat16), b.astype(jnp.bfloat16),
                   preferred_element_type=jnp.float32)


def _rowsq_sum(v, axis):
    sq = v * v
    if axis == 1:
        a = ((sq[:, 0:8] + sq[:, 8:16]) + sq[:, 16:24]) + sq[:, 24:32]
        c = lambda i: a[:, i:i + 1]
    else:
        a = ((sq[0:8, :] + sq[8:16, :]) + sq[16:24, :]) + sq[24:32, :]
        c = lambda i: a[i:i + 1, :]
    return (((c(0) + c(4)) + (c(2) + c(6)))
            + ((c(1) + c(5)) + (c(3) + c(7))))


def _enc_body(x_ref, we1_ref, be1_ref, we2_ref, be2_ref, we3_ref, be3_ref,
              embt_ref, ze_ref, idx_ref):
    g1 = _gelu(_dot_bf(x_ref[...], we1_ref[...]) + be1_ref[...])
    g2 = _gelu(_dot_bf(g1, we2_ref[...]) + be2_ref[...])
    ze = _dot_bf(g2, we3_ref[...]) + be3_ref[...]
    ze_ref[...] = ze
    zsq = _rowsq_sum(ze, axis=1)
    zeb = ze.astype(jnp.bfloat16)

    tms, tis = [], []
    for t in range(NC):
        w = embt_ref[:, t * KC:(t + 1) * KC]
        esq = _f32(0.25) * _rowsq_sum(w, axis=0)
        dot2 = jnp.dot(zeb, w.astype(jnp.bfloat16),
                       preferred_element_type=jnp.float32)
        dist = (zsq + esq) + dot2
        tmin = jnp.min(dist, axis=1, keepdims=True)
        iota = jax.lax.broadcasted_iota(jnp.int32, (BM, KC), 1) + (t * KC)
        tidx = jnp.min(jnp.where(dist == tmin, iota, jnp.int32(K)),
                       axis=1, keepdims=True)
        tms.append(tmin)
        tis.append(tidx)
    a_take1 = tms[1] < tms[0]
    a_v = jnp.where(a_take1, tms[1], tms[0])
    a_i = jnp.where(a_take1, tis[1], tis[0])
    b_take1 = tms[3] < tms[2]
    b_v = jnp.where(b_take1, tms[3], tms[2])
    b_i = jnp.where(b_take1, tis[3], tis[2])
    take_b = b_v < a_v.astype(jnp.bfloat16).astype(jnp.float32)
    idx_ref[...] = jnp.where(take_b, b_i, a_i)


def _encode_argmin(x, we1, be1, we2, be2, we3, be3, embt):
    return pl.pallas_call(
        _enc_body,
        grid=(NB,),
        in_specs=[
            pl.BlockSpec((BM, D_IN), lambda i: (i, 0)),
            pl.BlockSpec((D_IN, H), lambda i: (0, 0)),
            pl.BlockSpec((1, H), lambda i: (0, 0)),
            pl.BlockSpec((H, H), lambda i: (0, 0)),
            pl.BlockSpec((1, H), lambda i: (0, 0)),
            pl.BlockSpec((H, L), lambda i: (0, 0)),
            pl.BlockSpec((1, L), lambda i: (0, 0)),
            pl.BlockSpec((L, K), lambda i: (0, 0)),
        ],
        out_specs=[
            pl.BlockSpec((BM, L), lambda i: (i, 0)),
            pl.BlockSpec((BM, 1), lambda i: (i, 0)),
        ],
        out_shape=[
            jax.ShapeDtypeStruct((B, L), jnp.float32),
            jax.ShapeDtypeStruct((B, 1), jnp.int32),
        ],
    )(x, we1, be1, we2, be2, we3, be3, embt)


def _sc_gather(embed_pad, idx2):
    mesh = plsc.VectorSubcoreMesh(core_axis_name="core",
                                  subcore_axis_name="subcore")

    @pl.kernel(out_type=jax.ShapeDtypeStruct((B, 128), jnp.float32), mesh=mesh)
    def gather_kernel(emb_hbm, i_hbm, o_hbm):
        def body(i_vmem, o_vmem):
            pltpu.sync_copy(emb_hbm.at[i_vmem.at[0]], o_vmem)

        pltpu.emit_pipeline(
            body,
            grid=(B // GW,),
            in_specs=[pl.BlockSpec((1, GW), index_map=lambda i: (0, i))],
            out_specs=[pl.BlockSpec((GW, 128), index_map=lambda i: (i, 0))],
            core_axis_name=("core", "subcore"),
            dimension_semantics=(pltpu.PARALLEL,),
        )(i_hbm, o_hbm)

    return gather_kernel(embed_pad, idx2)


def _dec_body(zq_ref, ze_ref, idx_in_ref, wd1_ref, bd1_ref, wd2_ref, bd2_ref,
              wd3_ref, bd3_ref, xr_ref, part_ref, idx_out_ref):
    zq = zq_ref[:, :L]
    d = _gelu(_dot_bf(zq, wd1_ref[...]) + bd1_ref[...])
    d = _gelu(_dot_bf(d, wd2_ref[...]) + bd2_ref[...])
    xr_ref[...] = _dot_bf(d, wd3_ref[...]) + bd3_ref[...]
    diff = ze_ref[...] - zq
    part_ref[...] = jnp.sum(diff * diff, axis=0, keepdims=True)[None]
    idx_out_ref[...] = idx_in_ref[...]


def _decode(zq, ze, idx2, wd1, bd1, wd2, bd2, wd3, bd3):
    return pl.pallas_call(
        _dec_body,
        grid=(NB,),
        in_specs=[
            pl.BlockSpec((BM, 128), lambda i: (i, 0)),
            pl.BlockSpec((BM, L), lambda i: (i, 0)),
            pl.BlockSpec((BM, 1), lambda i: (i, 0)),
            pl.BlockSpec((L, H), lambda i: (0, 0)),
            pl.BlockSpec((1, H), lambda i: (0, 0)),
            pl.BlockSpec((H, H), lambda i: (0, 0)),
            pl.BlockSpec((1, H), lambda i: (0, 0)),
            pl.BlockSpec((H, D_IN), lambda i: (0, 0)),
            pl.BlockSpec((1, D_IN), lambda i: (0, 0)),
        ],
        out_specs=[
            pl.BlockSpec((BM, D_IN), lambda i: (i, 0)),
            pl.BlockSpec((1, 1, L), lambda i: (i, 0, 0)),
            pl.BlockSpec((BM, 1), lambda i: (i, 0)),
        ],
        out_shape=[
            jax.ShapeDtypeStruct((B, D_IN), jnp.float32),
            jax.ShapeDtypeStruct((NB, 1, L), jnp.float32),
            jax.ShapeDtypeStruct((B, 1), jnp.int32),
        ],
    )(zq, ze, idx2, wd1, bd1, wd2, bd2, wd3, bd3)


def kernel(x, embed, W_e1, b_e1, W_e2, b_e2, W_e3, b_e3,
           W_d1, b_d1, W_d2, b_d2, W_d3, b_d3):
    embt = jnp.transpose(_f32(-2.0) * embed)
    ze, idx2 = _encode_argmin(
        x, W_e1, b_e1.reshape(1, H), W_e2, b_e2.reshape(1, H),
        W_e3, b_e3.reshape(1, L), embt)
    embed_pad = jnp.pad(embed, ((0, 0), (0, 128 - L)))
    zq = _sc_gather(embed_pad, idx2.reshape(1, B))
    x_recon, parts, idx_out = _decode(
        zq, ze, idx2, W_d1, b_d1.reshape(1, H), W_d2, b_d2.reshape(1, H),
        W_d3, b_d3.reshape(1, D_IN))
    vq_loss = jnp.sum(parts) * (1.25 / (B * L))
    return (x_recon, vq_loss, idx_out.reshape(B))
